# baseline (speedup 1.0000x reference)
import dataclasses
import functools

import jax
import jax.numpy as jnp
from jax import lax
from jax.experimental import pallas as pl
from jax.experimental.pallas import tpu as pltpu
from jax.experimental.pallas import tpu_sc as plsc

CUTOFF = 10.0
CUTOFF_SR = 2.0
KEHALF = 7.199822675975274
N_NODES = 100000
N_EDGES = 6400000

NUM_CORES = 2
NUM_SUBCORES = 16
NW = NUM_CORES * NUM_SUBCORES
EDGES_PER_W = N_EDGES // NW
CHUNK = 2000
NCHUNK = EDGES_PER_W // CHUNK
L = 16

_A_COEFFS = (
    -0.623133386018961,
    10.975398522226984,
    -3.8909541280080018,
    -1.208226322526596,
    1.0907324878393378,
    -0.19851672339240242,
)


def _edge_body(d, qu, qv):
    p = jnp.full((L,), _A_COEFFS[-1], jnp.float32)
    for c in _A_COEFFS[-2::-1]:
        p = p * d + jnp.float32(c)
    a = jnp.where(d < jnp.float32(CUTOFF_SR), p, jnp.float32(KEHALF))
    chi = a / d
    f = chi - (jnp.float32(KEHALF * 2.0 / CUTOFF)
               - jnp.float32(KEHALF / (CUTOFF * CUTOFF)) * d)
    e = qu * qv * f
    return jnp.where(d <= jnp.float32(CUTOFF), e, jnp.float32(0.0))


def kernel(distances_uv, atomic_charges, idx_u, idx_v):
    idx_u = idx_u.astype(jnp.int32)
    idx_v = idx_v.astype(jnp.int32)
    mesh = plsc.VectorSubcoreMesh(core_axis_name="c", subcore_axis_name="s")

    cp = pltpu.CompilerParams()
    if "needs_layout_passes" in pltpu.CompilerParams.__dataclass_fields__:
        cp = dataclasses.replace(cp, needs_layout_passes=False)

    @functools.partial(
        pl.kernel,
        mesh=mesh,
        out_type=jax.ShapeDtypeStruct((N_EDGES,), jnp.float32),
        scratch_types=[
            pltpu.VMEM((N_NODES,), jnp.float32),
            pltpu.VMEM((CHUNK,), jnp.int32),
            pltpu.VMEM((CHUNK,), jnp.int32),
            pltpu.VMEM((CHUNK,), jnp.int32),
            pltpu.VMEM((CHUNK,), jnp.int32),
            pltpu.VMEM((CHUNK,), jnp.float32),
            pltpu.VMEM((CHUNK,), jnp.float32),
            pltpu.VMEM((CHUNK,), jnp.float32),
            pltpu.VMEM((CHUNK,), jnp.float32),
            pltpu.SemaphoreType.DMA,
            pltpu.SemaphoreType.DMA,
            pltpu.SemaphoreType.DMA,
            pltpu.SemaphoreType.DMA,
            pltpu.SemaphoreType.DMA,
        ],
        compiler_params=cp,
    )
    def run(d_hbm, q_hbm, iu_hbm, iv_hbm, out_hbm, q_v,
            iu_v0, iu_v1, iv_v0, iv_v1, d_v0, d_v1, o_v0, o_v1,
            sem_q, sem_in0, sem_in1, sem_out0, sem_out1):
        wid = lax.axis_index("s") * NUM_CORES + lax.axis_index("c")
        base = wid * EDGES_PER_W
        iu_v = (iu_v0, iu_v1)
        iv_v = (iv_v0, iv_v1)
        d_v = (d_v0, d_v1)
        o_v = (o_v0, o_v1)
        sem_in = (sem_in0, sem_in1)
        sem_out = (sem_out0, sem_out1)

        def fire_in(ci, b):
            off = base + ci * CHUNK
            pltpu.async_copy(iu_hbm.at[pl.ds(off, CHUNK)], iu_v[b], sem_in[b])
            pltpu.async_copy(iv_hbm.at[pl.ds(off, CHUNK)], iv_v[b], sem_in[b])
            pltpu.async_copy(d_hbm.at[pl.ds(off, CHUNK)], d_v[b], sem_in[b])

        def wait_in(b):
            pltpu.make_async_copy(iu_hbm.at[pl.ds(base, CHUNK)], iu_v[b], sem_in[b]).wait()
            pltpu.make_async_copy(iv_hbm.at[pl.ds(base, CHUNK)], iv_v[b], sem_in[b]).wait()
            pltpu.make_async_copy(d_hbm.at[pl.ds(base, CHUNK)], d_v[b], sem_in[b]).wait()

        pltpu.async_copy(q_hbm, q_v, sem_q)
        for b in range(2):
            fire_in(b, b)
        pltpu.make_async_copy(q_hbm, q_v, sem_q).wait()

        @pl.loop(0, NCHUNK, step=2)
        def _(ci):
            for b in range(2):
                cur = ci + b
                off = base + cur * CHUNK
                wait_in(b)

                @pl.when(cur >= 2)
                def _():
                    pltpu.make_async_copy(
                        o_v[b], out_hbm.at[pl.ds(base, CHUNK)], sem_out[b]
                    ).wait()

                @pl.loop(0, CHUNK, step=L)
                def _(j):
                    d = d_v[b][pl.ds(j, L)]
                    iu = iu_v[b][pl.ds(j, L)]
                    iv = iv_v[b][pl.ds(j, L)]
                    qu = plsc.load_gather(q_v, [iu])
                    qv = plsc.load_gather(q_v, [iv])
                    o_v[b][pl.ds(j, L)] = _edge_body(d, qu, qv)

                pltpu.async_copy(o_v[b], out_hbm.at[pl.ds(off, CHUNK)], sem_out[b])

                @pl.when(cur + 2 < NCHUNK)
                def _():
                    fire_in(cur + 2, b)

        for b in range(2):
            pltpu.make_async_copy(
                o_v[b], out_hbm.at[pl.ds(base, CHUNK)], sem_out[b]
            ).wait()

    return run(distances_uv, atomic_charges, idx_u, idx_v)
